# baseline (speedup 1.0000x reference)
import jax
import jax.numpy as jnp
from jax import lax
from jax.experimental import pallas as pl
from jax.experimental.pallas import tpu as pltpu
from jax.experimental.pallas import tpu_sc as plsc

D_MODEL = 1024
D_EXPERT = 1024
E = 8
K = 2
BLK = 512
RB = 512
LANES = 128
NC = 2
NS = 16
NW = NC * NS
CH = 64
CCH = 16


def _router_body(x_ref, rw_ref, i1_ref, i2_ref, p1_ref, p2_ref,
                 r1_ref, r2_ref, cnt_ref, x16_ref, carry):
    @pl.when(pl.program_id(0) == 0)
    def _():
        carry[...] = jnp.zeros((8, LANES), jnp.float32)

    x = x_ref[...]
    xb = x.astype(jnp.bfloat16)
    lo = jax.lax.bitcast_convert_type(xb[:, : D_MODEL // 2], jnp.uint16)
    hi = jax.lax.bitcast_convert_type(xb[:, D_MODEL // 2 :], jnp.uint16)
    packed = (hi.astype(jnp.uint32) << 16) | lo.astype(jnp.uint32)
    x16_ref[...] = jax.lax.bitcast_convert_type(packed, jnp.int32)
    logits = jnp.dot(x, rw_ref[...], preferred_element_type=jnp.float32)
    lane = lax.broadcasted_iota(jnp.int32, (RB, LANES), 1)
    neg = jnp.float32(-jnp.inf)
    l = jnp.where(lane < E, logits, neg)
    m1 = jnp.max(l, axis=1, keepdims=True)
    i1 = jnp.min(jnp.where(l == m1, lane, LANES), axis=1, keepdims=True)
    l2 = jnp.where(lane == i1, neg, l)
    m2 = jnp.max(l2, axis=1, keepdims=True)
    i2 = jnp.min(jnp.where(l2 == m2, lane, LANES), axis=1, keepdims=True)
    e2 = jnp.exp(m2 - m1)
    denom = 1.0 + e2
    i1_ref[...] = i1
    i2_ref[...] = i2
    p1_ref[...] = jnp.broadcast_to(1.0 / denom, (RB, 16))
    p2_ref[...] = jnp.broadcast_to(e2 / denom, (RB, 16))

    oh = (lane == i1).astype(jnp.float32) + (lane == i2).astype(jnp.float32)
    row = lax.broadcasted_iota(jnp.int32, (RB, RB), 0)
    col = lax.broadcasted_iota(jnp.int32, (RB, RB), 1)
    tril = (row > col).astype(jnp.float32)
    excl = jnp.dot(tril, oh, preferred_element_type=jnp.float32)
    rank_rows = excl + carry[0:1, :]
    r1_ref[...] = jnp.sum(jnp.where(lane == i1, rank_rows, 0.0),
                          axis=1, keepdims=True).astype(jnp.int32)
    r2_ref[...] = jnp.sum(jnp.where(lane == i2, rank_rows, 0.0),
                          axis=1, keepdims=True).astype(jnp.int32)
    new_carry = carry[0:1, :] + jnp.sum(oh, axis=0, keepdims=True)
    carry[0:1, :] = new_carry
    cnt_ref[...] = new_carry


QB = 128


def _mm_body(be_ref, vr_ref, xs_ref, w1_ref, w3_ref, w2_ref, ys_ref):
    g = pl.program_id(0)
    vr = vr_ref[g]

    for q in range(BLK // QB):
        @pl.when(vr > q * QB)
        def _(q=q):
            xu = jax.lax.bitcast_convert_type(
                xs_ref[q * QB:(q + 1) * QB, :], jnp.uint32)
            lo = jax.lax.bitcast_convert_type(
                (xu & jnp.uint32(0xFFFF)).astype(jnp.uint16), jnp.bfloat16)
            hi = jax.lax.bitcast_convert_type(
                (xu >> jnp.uint32(16)).astype(jnp.uint16), jnp.bfloat16)
            x = jnp.concatenate([lo, hi], axis=1).astype(jnp.float32)
            a = jnp.dot(x, w1_ref[0], preferred_element_type=jnp.float32)
            b = jnp.dot(x, w3_ref[0], preferred_element_type=jnp.float32)
            h = a * jax.nn.sigmoid(a) * b
            ys_ref[q * QB:(q + 1) * QB, :] = jnp.dot(
                h, w2_ref[0], preferred_element_type=jnp.float32)


def _dispatch_body(x_hbm, sa_hbm, sb_hbm, xs_hbm, rows_v, ia_v, ib_v, sg0, sg1):
    wid = lax.axis_index("s") * NC + lax.axis_index("c")
    tpw = x_hbm.shape[0] // NW
    base = wid * tpw
    nch = tpw // CH
    sems = (sg0, sg1)

    def issue(c, b):
        off = base + c * CH
        pltpu.sync_copy(x_hbm.at[pl.ds(off, CH)], rows_v.at[b])
        pltpu.sync_copy(sa_hbm.at[pl.ds(off, CH)], ia_v.at[b])
        pltpu.sync_copy(sb_hbm.at[pl.ds(off, CH)], ib_v.at[b])
        ca = pltpu.async_copy(rows_v.at[b], xs_hbm.at[ia_v.at[b]], sems[b])
        cb = pltpu.async_copy(rows_v.at[b], xs_hbm.at[ib_v.at[b]], sems[b])
        return ca, cb

    pend = {0: issue(0, 0), 1: None}
    for c in range(nch):
        b = c % 2
        if c + 1 < nch:
            pend[1 - b] = issue(c + 1, 1 - b)
        ca, cb = pend[b]
        ca.wait()
        cb.wait()


def _combine_body(ys_hbm, sa_hbm, sb_hbm, pa_hbm, pb_hbm, out_hbm,
                  A_v, B_v, O_v, ia_v, ib_v, pa_v, pb_v, sg0, sg1):
    wid = lax.axis_index("s") * NC + lax.axis_index("c")
    tpw = out_hbm.shape[0] // NW
    base = wid * tpw
    nch = tpw // CCH
    sems = (sg0, sg1)

    def issue(c, b):
        off = base + c * CCH
        pltpu.sync_copy(sa_hbm.at[pl.ds(off, CCH)], ia_v.at[b])
        pltpu.sync_copy(sb_hbm.at[pl.ds(off, CCH)], ib_v.at[b])
        pltpu.sync_copy(pa_hbm.at[pl.ds(off, CCH)], pa_v.at[b])
        pltpu.sync_copy(pb_hbm.at[pl.ds(off, CCH)], pb_v.at[b])
        ca = pltpu.async_copy(ys_hbm.at[ia_v.at[b]], A_v.at[b], sems[b])
        cb = pltpu.async_copy(ys_hbm.at[ib_v.at[b]], B_v.at[b], sems[b])
        return ca, cb

    pend = {0: issue(0, 0), 1: None}
    for c in range(nch):
        b = c % 2
        if c + 1 < nch:
            pend[1 - b] = issue(c + 1, 1 - b)
        ca, cb = pend[b]
        ca.wait()
        cb.wait()

        def rowfn(r, _, b=b):
            wa = pa_v[b, r, :]
            wb = pb_v[b, r, :]

            def colfn(j):
                for u in range(4):
                    s = pl.ds(j * 64 + u * 16, 16)
                    O_v[r, s] = A_v[b, r, s] * wa + B_v[b, r, s] * wb

            plsc.parallel_loop(0, D_MODEL // 64, unroll=4)(colfn)
            return 0

        lax.fori_loop(0, CCH, rowfn, 0)
        pltpu.sync_copy(O_v, out_hbm.at[pl.ds(base + c * CCH, CCH)])


def kernel(x, router_w, w1, w2, w3):
    B, T, C = x.shape
    N = B * T
    NK = N * K
    G = NK // BLK + E
    S = G * BLK
    x_flat = x.reshape(N, C)

    rw_pad = jnp.pad(router_w, ((0, 0), (0, LANES - E)))
    i1, i2, p1, p2, r1, r2, cnt, x16 = pl.pallas_call(
        _router_body,
        grid=(N // RB,),
        in_specs=[
            pl.BlockSpec((RB, C), lambda i: (i, 0)),
            pl.BlockSpec((C, LANES), lambda i: (0, 0)),
        ],
        out_specs=[
            pl.BlockSpec((RB, 1), lambda i: (i, 0)),
            pl.BlockSpec((RB, 1), lambda i: (i, 0)),
            pl.BlockSpec((RB, 16), lambda i: (i, 0)),
            pl.BlockSpec((RB, 16), lambda i: (i, 0)),
            pl.BlockSpec((RB, 1), lambda i: (i, 0)),
            pl.BlockSpec((RB, 1), lambda i: (i, 0)),
            pl.BlockSpec((1, LANES), lambda i: (0, 0)),
            pl.BlockSpec((RB, C // 2), lambda i: (i, 0)),
        ],
        out_shape=[
            jax.ShapeDtypeStruct((N, 1), jnp.int32),
            jax.ShapeDtypeStruct((N, 1), jnp.int32),
            jax.ShapeDtypeStruct((N, 16), jnp.float32),
            jax.ShapeDtypeStruct((N, 16), jnp.float32),
            jax.ShapeDtypeStruct((N, 1), jnp.int32),
            jax.ShapeDtypeStruct((N, 1), jnp.int32),
            jax.ShapeDtypeStruct((1, LANES), jnp.float32),
            jax.ShapeDtypeStruct((N, C // 2), jnp.int32),
        ],
        scratch_shapes=[pltpu.VMEM((8, LANES), jnp.float32)],
    )(x_flat, rw_pad)

    counts = cnt[0, :E].astype(jnp.int32)
    padded = ((counts + BLK - 1) // BLK) * BLK
    er = jnp.arange(E, dtype=jnp.int32)
    ends = jnp.sum(jnp.where(er[:, None] >= er[None, :], padded[None, :], 0), axis=1)
    pad_off = ends - padded
    sa = (pad_off[i1[:, 0]] + r1[:, 0]).astype(jnp.int32)
    sb = (pad_off[i2[:, 0]] + r2[:, 0]).astype(jnp.int32)
    gblk = jnp.arange(G, dtype=jnp.int32) * BLK
    be = jnp.minimum(
        jnp.sum((gblk[:, None] >= ends[None, :]).astype(jnp.int32), axis=1),
        E - 1).astype(jnp.int32)
    vr = jnp.clip((pad_off[be] + counts[be]) - gblk, 0, BLK).astype(jnp.int32)

    mesh = plsc.VectorSubcoreMesh(core_axis_name="c", subcore_axis_name="s")

    xs = pl.kernel(
        _dispatch_body,
        out_type=jax.ShapeDtypeStruct((S, C // 2), jnp.int32),
        mesh=mesh,
        scratch_types=[
            pltpu.VMEM((2, CH, C // 2), jnp.int32),
            pltpu.VMEM((2, CH), jnp.int32),
            pltpu.VMEM((2, CH), jnp.int32),
            pltpu.SemaphoreType.DMA,
            pltpu.SemaphoreType.DMA,
        ],
    )(x16, sa, sb)

    grid_spec = pltpu.PrefetchScalarGridSpec(
        num_scalar_prefetch=2,
        grid=(G,),
        in_specs=[
            pl.BlockSpec((BLK, C // 2), lambda g, be, used: (g, 0)),
            pl.BlockSpec((1, C, D_EXPERT), lambda g, be, used: (be[g], 0, 0)),
            pl.BlockSpec((1, C, D_EXPERT), lambda g, be, used: (be[g], 0, 0)),
            pl.BlockSpec((1, D_EXPERT, C), lambda g, be, used: (be[g], 0, 0)),
        ],
        out_specs=pl.BlockSpec((BLK, C), lambda g, be, used: (g, 0)),
    )
    ys = pl.pallas_call(
        _mm_body,
        grid_spec=grid_spec,
        out_shape=jax.ShapeDtypeStruct((S, C), jnp.float32),
    )(be, vr, xs, w1, w3, w2)

    out = pl.kernel(
        _combine_body,
        out_type=jax.ShapeDtypeStruct((N, C), jnp.float32),
        mesh=mesh,
        scratch_types=[
            pltpu.VMEM((2, CCH, C), jnp.float32),
            pltpu.VMEM((2, CCH, C), jnp.float32),
            pltpu.VMEM((CCH, C), jnp.float32),
            pltpu.VMEM((2, CCH), jnp.int32),
            pltpu.VMEM((2, CCH), jnp.int32),
            pltpu.VMEM((2, CCH, 16), jnp.float32),
            pltpu.VMEM((2, CCH, 16), jnp.float32),
            pltpu.SemaphoreType.DMA,
            pltpu.SemaphoreType.DMA,
        ],
    )(ys, sa, sb, p1, p2)

    return out.reshape(B, T, C)


# speedup vs baseline: 1.0233x; 1.0233x over previous
import jax
import jax.numpy as jnp
from jax import lax
from jax.experimental import pallas as pl
from jax.experimental.pallas import tpu as pltpu
from jax.experimental.pallas import tpu_sc as plsc

D_MODEL = 1024
D_EXPERT = 1024
E = 8
K = 2
BLK = 512
RB = 512
LANES = 128
NC = 2
NS = 16
NW = NC * NS
CH = 64
CCH = 16


def _router_body(x_ref, rw_ref, i1_ref, i2_ref, p1_ref, p2_ref,
                 r1_ref, r2_ref, cnt_ref, x16_ref, carry):
    @pl.when(pl.program_id(0) == 0)
    def _():
        carry[...] = jnp.zeros((8, LANES), jnp.float32)

    x = x_ref[...]
    xb = x.astype(jnp.bfloat16)
    lo = jax.lax.bitcast_convert_type(xb[:, : D_MODEL // 2], jnp.uint16)
    hi = jax.lax.bitcast_convert_type(xb[:, D_MODEL // 2 :], jnp.uint16)
    packed = (hi.astype(jnp.uint32) << 16) | lo.astype(jnp.uint32)
    x16_ref[...] = jax.lax.bitcast_convert_type(packed, jnp.int32)
    logits = jnp.dot(x, rw_ref[...], preferred_element_type=jnp.float32)
    lane = lax.broadcasted_iota(jnp.int32, (RB, LANES), 1)
    neg = jnp.float32(-jnp.inf)
    l = jnp.where(lane < E, logits, neg)
    m1 = jnp.max(l, axis=1, keepdims=True)
    i1 = jnp.min(jnp.where(l == m1, lane, LANES), axis=1, keepdims=True)
    l2 = jnp.where(lane == i1, neg, l)
    m2 = jnp.max(l2, axis=1, keepdims=True)
    i2 = jnp.min(jnp.where(l2 == m2, lane, LANES), axis=1, keepdims=True)
    e2 = jnp.exp(m2 - m1)
    denom = 1.0 + e2
    i1_ref[...] = i1
    i2_ref[...] = i2
    p1_ref[...] = jnp.broadcast_to(1.0 / denom, (RB, 16))
    p2_ref[...] = jnp.broadcast_to(e2 / denom, (RB, 16))

    oh = (lane == i1).astype(jnp.float32) + (lane == i2).astype(jnp.float32)
    row = lax.broadcasted_iota(jnp.int32, (RB, RB), 0)
    col = lax.broadcasted_iota(jnp.int32, (RB, RB), 1)
    tril = (row > col).astype(jnp.float32)
    excl = jnp.dot(tril, oh, preferred_element_type=jnp.float32)
    rank_rows = excl + carry[0:1, :]
    r1_ref[...] = jnp.sum(jnp.where(lane == i1, rank_rows, 0.0),
                          axis=1, keepdims=True).astype(jnp.int32)
    r2_ref[...] = jnp.sum(jnp.where(lane == i2, rank_rows, 0.0),
                          axis=1, keepdims=True).astype(jnp.int32)
    new_carry = carry[0:1, :] + jnp.sum(oh, axis=0, keepdims=True)
    carry[0:1, :] = new_carry
    cnt_ref[...] = new_carry


def _mm_body(be_ref, vr_ref, xs_ref, w1_ref, w3_ref, w2_ref, ys_ref):
    g = pl.program_id(0)

    @pl.when(vr_ref[g] > 0)
    def _():
        xu = jax.lax.bitcast_convert_type(xs_ref[...], jnp.uint32)
        lo = jax.lax.bitcast_convert_type(
            (xu & jnp.uint32(0xFFFF)).astype(jnp.uint16), jnp.bfloat16)
        hi = jax.lax.bitcast_convert_type(
            (xu >> jnp.uint32(16)).astype(jnp.uint16), jnp.bfloat16)
        x = jnp.concatenate([lo, hi], axis=1).astype(jnp.float32)
        a = jnp.dot(x, w1_ref[0], preferred_element_type=jnp.float32)
        b = jnp.dot(x, w3_ref[0], preferred_element_type=jnp.float32)
        h = a * jax.nn.sigmoid(a) * b
        ys_ref[...] = jnp.dot(h, w2_ref[0], preferred_element_type=jnp.float32)


def _dispatch_body(x_hbm, sa_hbm, sb_hbm, xs_hbm, rows_v, ia_v, ib_v, sg0, sg1):
    wid = lax.axis_index("s") * NC + lax.axis_index("c")
    tpw = x_hbm.shape[0] // NW
    base = wid * tpw
    nch = tpw // CH
    sems = (sg0, sg1)

    def issue(c, b):
        off = base + c * CH
        pltpu.sync_copy(x_hbm.at[pl.ds(off, CH)], rows_v.at[b])
        pltpu.sync_copy(sa_hbm.at[pl.ds(off, CH)], ia_v.at[b])
        pltpu.sync_copy(sb_hbm.at[pl.ds(off, CH)], ib_v.at[b])
        ca = pltpu.async_copy(rows_v.at[b], xs_hbm.at[ia_v.at[b]], sems[b])
        cb = pltpu.async_copy(rows_v.at[b], xs_hbm.at[ib_v.at[b]], sems[b])
        return ca, cb

    pend = {0: issue(0, 0), 1: None}
    for c in range(nch):
        b = c % 2
        if c + 1 < nch:
            pend[1 - b] = issue(c + 1, 1 - b)
        ca, cb = pend[b]
        ca.wait()
        cb.wait()


def _combine_body(ys_hbm, sa_hbm, sb_hbm, pa_hbm, pb_hbm, out_hbm,
                  A_v, B_v, O_v, ia_v, ib_v, pa_v, pb_v, sg0, sg1):
    wid = lax.axis_index("s") * NC + lax.axis_index("c")
    tpw = out_hbm.shape[0] // NW
    base = wid * tpw
    nch = tpw // CCH
    sems = (sg0, sg1)

    def issue(c, b):
        off = base + c * CCH
        pltpu.sync_copy(sa_hbm.at[pl.ds(off, CCH)], ia_v.at[b])
        pltpu.sync_copy(sb_hbm.at[pl.ds(off, CCH)], ib_v.at[b])
        pltpu.sync_copy(pa_hbm.at[pl.ds(off, CCH)], pa_v.at[b])
        pltpu.sync_copy(pb_hbm.at[pl.ds(off, CCH)], pb_v.at[b])
        ca = pltpu.async_copy(ys_hbm.at[ia_v.at[b]], A_v.at[b], sems[b])
        cb = pltpu.async_copy(ys_hbm.at[ib_v.at[b]], B_v.at[b], sems[b])
        return ca, cb

    pend = {0: issue(0, 0), 1: None}
    for c in range(nch):
        b = c % 2
        if c + 1 < nch:
            pend[1 - b] = issue(c + 1, 1 - b)
        ca, cb = pend[b]
        ca.wait()
        cb.wait()

        def rowfn(r, _, b=b):
            wa = pa_v[b, r, :]
            wb = pb_v[b, r, :]

            def colfn(j):
                for u in range(4):
                    s = pl.ds(j * 64 + u * 16, 16)
                    O_v[r, s] = A_v[b, r, s] * wa + B_v[b, r, s] * wb

            plsc.parallel_loop(0, D_MODEL // 64, unroll=4)(colfn)
            return 0

        lax.fori_loop(0, CCH, rowfn, 0)
        pltpu.sync_copy(O_v, out_hbm.at[pl.ds(base + c * CCH, CCH)])


def kernel(x, router_w, w1, w2, w3):
    B, T, C = x.shape
    N = B * T
    NK = N * K
    G = NK // BLK + E
    S = G * BLK
    x_flat = x.reshape(N, C)

    rw_pad = jnp.pad(router_w, ((0, 0), (0, LANES - E)))
    i1, i2, p1, p2, r1, r2, cnt, x16 = pl.pallas_call(
        _router_body,
        grid=(N // RB,),
        in_specs=[
            pl.BlockSpec((RB, C), lambda i: (i, 0)),
            pl.BlockSpec((C, LANES), lambda i: (0, 0)),
        ],
        out_specs=[
            pl.BlockSpec((RB, 1), lambda i: (i, 0)),
            pl.BlockSpec((RB, 1), lambda i: (i, 0)),
            pl.BlockSpec((RB, 16), lambda i: (i, 0)),
            pl.BlockSpec((RB, 16), lambda i: (i, 0)),
            pl.BlockSpec((RB, 1), lambda i: (i, 0)),
            pl.BlockSpec((RB, 1), lambda i: (i, 0)),
            pl.BlockSpec((1, LANES), lambda i: (0, 0)),
            pl.BlockSpec((RB, C // 2), lambda i: (i, 0)),
        ],
        out_shape=[
            jax.ShapeDtypeStruct((N, 1), jnp.int32),
            jax.ShapeDtypeStruct((N, 1), jnp.int32),
            jax.ShapeDtypeStruct((N, 16), jnp.float32),
            jax.ShapeDtypeStruct((N, 16), jnp.float32),
            jax.ShapeDtypeStruct((N, 1), jnp.int32),
            jax.ShapeDtypeStruct((N, 1), jnp.int32),
            jax.ShapeDtypeStruct((1, LANES), jnp.float32),
            jax.ShapeDtypeStruct((N, C // 2), jnp.int32),
        ],
        scratch_shapes=[pltpu.VMEM((8, LANES), jnp.float32)],
    )(x_flat, rw_pad)

    counts = cnt[0, :E].astype(jnp.int32)
    padded = ((counts + BLK - 1) // BLK) * BLK
    er = jnp.arange(E, dtype=jnp.int32)
    ends = jnp.sum(jnp.where(er[:, None] >= er[None, :], padded[None, :], 0), axis=1)
    pad_off = ends - padded
    sa = (pad_off[i1[:, 0]] + r1[:, 0]).astype(jnp.int32)
    sb = (pad_off[i2[:, 0]] + r2[:, 0]).astype(jnp.int32)
    gblk = jnp.arange(G, dtype=jnp.int32) * BLK
    be = jnp.minimum(
        jnp.sum((gblk[:, None] >= ends[None, :]).astype(jnp.int32), axis=1),
        E - 1).astype(jnp.int32)
    vr = jnp.clip((pad_off[be] + counts[be]) - gblk, 0, BLK).astype(jnp.int32)

    mesh = plsc.VectorSubcoreMesh(core_axis_name="c", subcore_axis_name="s")

    xs = pl.kernel(
        _dispatch_body,
        out_type=jax.ShapeDtypeStruct((S, C // 2), jnp.int32),
        mesh=mesh,
        scratch_types=[
            pltpu.VMEM((2, CH, C // 2), jnp.int32),
            pltpu.VMEM((2, CH), jnp.int32),
            pltpu.VMEM((2, CH), jnp.int32),
            pltpu.SemaphoreType.DMA,
            pltpu.SemaphoreType.DMA,
        ],
    )(x16, sa, sb)

    grid_spec = pltpu.PrefetchScalarGridSpec(
        num_scalar_prefetch=2,
        grid=(G,),
        in_specs=[
            pl.BlockSpec((BLK, C // 2), lambda g, be, used: (g, 0)),
            pl.BlockSpec((1, C, D_EXPERT), lambda g, be, used: (be[g], 0, 0)),
            pl.BlockSpec((1, C, D_EXPERT), lambda g, be, used: (be[g], 0, 0)),
            pl.BlockSpec((1, D_EXPERT, C), lambda g, be, used: (be[g], 0, 0)),
        ],
        out_specs=pl.BlockSpec((BLK, C), lambda g, be, used: (g, 0)),
    )
    ys = pl.pallas_call(
        _mm_body,
        grid_spec=grid_spec,
        out_shape=jax.ShapeDtypeStruct((S, C), jnp.float32),
    )(be, vr, xs, w1, w3, w2)

    out = pl.kernel(
        _combine_body,
        out_type=jax.ShapeDtypeStruct((N, C), jnp.float32),
        mesh=mesh,
        scratch_types=[
            pltpu.VMEM((2, CCH, C), jnp.float32),
            pltpu.VMEM((2, CCH, C), jnp.float32),
            pltpu.VMEM((CCH, C), jnp.float32),
            pltpu.VMEM((2, CCH), jnp.int32),
            pltpu.VMEM((2, CCH), jnp.int32),
            pltpu.VMEM((2, CCH, 16), jnp.float32),
            pltpu.VMEM((2, CCH, 16), jnp.float32),
            pltpu.SemaphoreType.DMA,
            pltpu.SemaphoreType.DMA,
        ],
    )(ys, sa, sb, p1, p2)

    return out.reshape(B, T, C)
